# baseline (speedup 1.0000x reference)
import functools

import jax
import jax.numpy as jnp
from jax import lax
from jax.experimental import pallas as pl
from jax.experimental.pallas import tpu as pltpu
from jax.experimental.pallas import tpu_sc as plsc

NUM_EMBEDDINGS = 100000
EMBED_DIM = 64
B_ROWS = 4096
B_COLS = 100
TOTAL = B_ROWS * B_COLS

_info = plsc.get_sparse_core_info()
NC, NS = _info.num_cores, _info.num_subcores
NW = NC * NS

CHUNK = 128
HALF = CHUNK // 2
NBUF = 5
PER_W = TOTAL // NW
N_CHUNKS = PER_W // CHUNK
N_GROUPS = N_CHUNKS // NBUF

_mesh = plsc.VectorSubcoreMesh(core_axis_name="c", subcore_axis_name="s")


@functools.partial(
    pl.kernel,
    mesh=_mesh,
    out_type=jax.ShapeDtypeStruct((TOTAL // 2, 2 * EMBED_DIM), jnp.float32),
    scratch_types=[
        pltpu.VMEM((PER_W,), jnp.int32),
        pltpu.VMEM((NBUF, CHUNK, EMBED_DIM), jnp.float32),
        pltpu.SemaphoreType.DMA((NBUF,)),
        pltpu.SemaphoreType.DMA((NBUF,)),
        pltpu.SemaphoreType.DMA((NBUF,)),
    ],
    compiler_params=pltpu.CompilerParams(use_tc_tiling_on_sc=False),
)
def _sc_gather(idx_hbm, table_hbm, out_hbm, idx_v, rows_v, sem_g, sem_l, sem_r):
    wid = lax.axis_index("s") * NC + lax.axis_index("c")
    base = wid * PER_W

    pltpu.sync_copy(idx_hbm.at[pl.ds(base, PER_W)], idx_v)

    def store_l(b, row):
        return pltpu.make_async_copy(
            rows_v.at[b, pl.ds(0, HALF)],
            out_hbm.at[pl.ds(row, HALF), pl.ds(0, EMBED_DIM)],
            sem_l.at[b],
        )

    def store_r(b, row):
        return pltpu.make_async_copy(
            rows_v.at[b, pl.ds(HALF, HALF)],
            out_hbm.at[pl.ds(row, HALF), pl.ds(EMBED_DIM, EMBED_DIM)],
            sem_r.at[b],
        )

    def body(g, carry):
        goff = g * NBUF * CHUNK
        for b in range(NBUF):

            @pl.when(g > 0)
            def _wait_out():
                store_l(b, base // 2).wait()
                store_r(b, base // 2).wait()

            pltpu.make_async_copy(
                table_hbm.at[idx_v.at[pl.ds(goff + b * CHUNK, CHUNK)]],
                rows_v.at[b],
                sem_g.at[b],
            ).start()
        for b in range(NBUF):
            off = goff + b * CHUNK
            pltpu.make_async_copy(
                table_hbm.at[idx_v.at[pl.ds(off, CHUNK)]],
                rows_v.at[b],
                sem_g.at[b],
            ).wait()
            row = (base + off) // 2
            store_l(b, row).start()
            store_r(b, row).start()
        return carry

    lax.fori_loop(0, N_GROUPS, body, 0)

    for b in range(NBUF):
        store_l(b, base // 2).wait()
        store_r(b, base // 2).wait()


def kernel(emb, table):
    idx = emb.reshape(TOTAL // CHUNK, HALF, 2).transpose(0, 2, 1).reshape(-1)
    out2 = _sc_gather(idx, table)
    return out2.reshape(B_ROWS, B_COLS, EMBED_DIM)
